# baseline (speedup 1.0000x reference)
import functools

import jax
import jax.numpy as jnp
from jax import lax
from jax.experimental import pallas as pl
from jax.experimental.pallas import tpu as pltpu
from jax.experimental.pallas import tpu_sc as plsc

VOCAB = 1000000
DIM = 32
B = 4096
L = 50
N = B * L

_info = plsc.get_sparse_core_info()
_NC = _info.num_cores
_NS = _info.num_subcores
_NW = _NC * _NS

_PER_W = N // _NW
_CHUNK = 1280
_NCHUNK = _PER_W // _CHUNK


def _gather_kernel(post_i, resp_i, wiki_i, table, post_o, resp_o, wiki_o,
                   idx_v, rows_v, sem):
    wid = lax.axis_index("s") * _NC + lax.axis_index("c")
    base = wid * _PER_W
    for idx_hbm, out_hbm in ((post_i, post_o), (resp_i, resp_o),
                             (wiki_i, wiki_o)):
        for c in range(_NCHUNK):
            off = base + c * _CHUNK
            pltpu.sync_copy(idx_hbm.at[pl.ds(off, _CHUNK)], idx_v)
            pltpu.async_copy(table.at[idx_v], rows_v, sem).wait()
            pltpu.sync_copy(rows_v, out_hbm.at[pl.ds(off, _CHUNK)])


@jax.jit
def _embed3(post_f, resp_f, wiki_f, table):
    mesh = plsc.VectorSubcoreMesh(core_axis_name="c", subcore_axis_name="s")
    out = jax.ShapeDtypeStruct((N, DIM), jnp.float32)
    return pl.kernel(
        _gather_kernel,
        mesh=mesh,
        out_type=(out, out, out),
        scratch_types=[
            pltpu.VMEM((_CHUNK,), jnp.int32),
            pltpu.VMEM((_CHUNK, DIM), jnp.float32),
            pltpu.SemaphoreType.DMA,
        ],
        compiler_params=pltpu.CompilerParams(use_tc_tiling_on_sc=False),
    )(post_f, resp_f, wiki_f, table)


def kernel(post, resp, wiki, table):
    post_e, resp_e, wiki_e = _embed3(
        post.reshape(N), resp.reshape(N), wiki.reshape(N), table)
    return (post_e.reshape(B, L, DIM),
            resp_e.reshape(B, L, DIM),
            wiki_e.reshape(B, L, DIM))


# speedup vs baseline: 1.0105x; 1.0105x over previous
import functools

import jax
import jax.numpy as jnp
from jax import lax
from jax.experimental import pallas as pl
from jax.experimental.pallas import tpu as pltpu
from jax.experimental.pallas import tpu_sc as plsc

VOCAB = 1000000
DIM = 32
B = 4096
L = 50
N = B * L

_info = plsc.get_sparse_core_info()
_NC = _info.num_cores
_NS = _info.num_subcores
_NW = _NC * _NS

_PER_W = N // _NW
_CHUNK = 1280
_NCHUNK = _PER_W // _CHUNK


def _gather_kernel(post_i, resp_i, wiki_i, table, post_o, resp_o, wiki_o,
                   idx_v, rows_v, sem_g, sem_w):
    wid = lax.axis_index("s") * _NC + lax.axis_index("c")
    base = wid * _PER_W
    streams = ((post_i, post_o), (resp_i, resp_o), (wiki_i, wiki_o))
    jobs = [(s, c) for s in range(3) for c in range(_NCHUNK)]
    n = len(jobs)

    def src(j):
        s, c = jobs[j]
        return streams[s][0].at[pl.ds(base + c * _CHUNK, _CHUNK)]

    def dst(j):
        s, c = jobs[j]
        return streams[s][1].at[pl.ds(base + c * _CHUNK, _CHUNK)]

    gat = [None, None]
    wrb = [None, None]
    for j in range(n):
        b = j % 2
        if wrb[b] is not None:
            wrb[b].wait()
        pltpu.sync_copy(src(j), idx_v.at[b])
        gat[b] = pltpu.async_copy(table.at[idx_v.at[b]], rows_v.at[b], sem_g)
        if j > 0:
            pb = (j - 1) % 2
            gat[pb].wait()
            wrb[pb] = pltpu.async_copy(rows_v.at[pb], dst(j - 1), sem_w)
    lb = (n - 1) % 2
    gat[lb].wait()
    wrb[lb] = pltpu.async_copy(rows_v.at[lb], dst(n - 1), sem_w)
    wrb[0].wait()
    wrb[1].wait()


@jax.jit
def _embed3(post_f, resp_f, wiki_f, table):
    mesh = plsc.VectorSubcoreMesh(core_axis_name="c", subcore_axis_name="s")
    out = jax.ShapeDtypeStruct((N, DIM), jnp.float32)
    return pl.kernel(
        _gather_kernel,
        mesh=mesh,
        out_type=(out, out, out),
        scratch_types=[
            pltpu.VMEM((2, _CHUNK), jnp.int32),
            pltpu.VMEM((2, _CHUNK, DIM), jnp.float32),
            pltpu.SemaphoreType.DMA,
            pltpu.SemaphoreType.DMA,
        ],
        compiler_params=pltpu.CompilerParams(use_tc_tiling_on_sc=False),
    )(post_f, resp_f, wiki_f, table)


def kernel(post, resp, wiki, table):
    post_e, resp_e, wiki_e = _embed3(
        post.reshape(N), resp.reshape(N), wiki.reshape(N), table)
    return (post_e.reshape(B, L, DIM),
            resp_e.reshape(B, L, DIM),
            wiki_e.reshape(B, L, DIM))


# speedup vs baseline: 1.3370x; 1.3231x over previous
import functools

import jax
import jax.numpy as jnp
from jax import lax
from jax.experimental import pallas as pl
from jax.experimental.pallas import tpu as pltpu
from jax.experimental.pallas import tpu_sc as plsc

VOCAB = 1000000
DIM = 32
B = 4096
L = 50

_info = plsc.get_sparse_core_info()
_NC = _info.num_cores
_NS = _info.num_subcores
_NW = _NC * _NS
_BW = B // _NW
_NL = 5
_NJ = L // _NL


def _transpose_chunk(rows2, tile1, dbase0):
    def body(g, carry):
        for u in range(4):
            r = g * 4 + u
            li = r >> 7
            bl = r & 127
            off0 = dbase0 + (li * 4096 + bl)
            v0 = rows2[r, pl.ds(0, 16)]
            v1 = rows2[r, pl.ds(16, 16)]
            plsc.store_scatter(tile1, [off0], v0)
            plsc.store_scatter(tile1, [off0 + 2048], v1)
        return carry
    lax.fori_loop(0, (_NL * _BW) // 4, body, 0)


def _gather_kernel(post_i, resp_i, wiki_i, table, post_o, resp_o, wiki_o,
                   idx_v, rows_a, rows_b, tile_a, tile_b, sem_g, sem_w):
    rows_v = (rows_a, rows_b)
    tile_v = (tile_a, tile_b)
    wid = lax.axis_index("s") * _NC + lax.axis_index("c")
    b0 = wid * _BW
    woff = wid * 1024
    lane16 = lax.iota(jnp.int32, 16)
    dbase0 = ((lane16 >> 3) << 10) + ((lane16 & 7) << 7)
    streams = ((post_i, post_o), (resp_i, resp_o), (wiki_i, wiki_o))
    jobs = [(s, l0) for s in range(3) for l0 in range(0, L, _NL)]
    n = len(jobs)

    def load_idx(j, b):
        s, l0 = jobs[j]
        pltpu.sync_copy(streams[s][0].at[pl.ds(l0, _NL), pl.ds(b0, _BW)],
                        idx_v.at[b])

    def fire_gathers(j, b):
        return [pltpu.async_copy(table.at[idx_v.at[b, li]],
                                 rows_v[b].at[pl.ds(li * _BW, _BW)], sem_g)
                for li in range(_NL)]

    def fire_writebacks(j, b):
        s, l0 = jobs[j]
        out = streams[s][1]
        return [pltpu.async_copy(
                    tile_v[b].at[pl.ds((li * 4 + dh) * 1024, 1024)],
                    out.at[l0 + li, dh, pl.ds(woff, 1024)], sem_w)
                for li in range(_NL) for dh in range(4)]

    gat = [None, None]
    wrb = [None, None]
    for j in range(n):
        b = j & 1
        if wrb[b] is not None:
            for cp in wrb[b]:
                cp.wait()
            wrb[b] = None
        load_idx(j, b)
        gat[b] = fire_gathers(j, b)
        if j > 0:
            pb = (j - 1) & 1
            for cp in gat[pb]:
                cp.wait()
            _transpose_chunk(rows_v[pb], tile_v[pb], dbase0)
            wrb[pb] = fire_writebacks(j - 1, pb)
    lb = (n - 1) & 1
    for cp in gat[lb]:
        cp.wait()
    _transpose_chunk(rows_v[lb], tile_v[lb], dbase0)
    for cp in fire_writebacks(n - 1, lb):
        cp.wait()
    for cp in wrb[1 - lb] or ():
        cp.wait()


@jax.jit
def _embed3(post_t, resp_t, wiki_t, table):
    mesh = plsc.VectorSubcoreMesh(core_axis_name="c", subcore_axis_name="s")
    out = jax.ShapeDtypeStruct((L, 4, _NW * 1024), jnp.float32)
    return pl.kernel(
        _gather_kernel,
        mesh=mesh,
        out_type=(out, out, out),
        scratch_types=[
            pltpu.VMEM((2, _NL, _BW), jnp.int32),
            pltpu.VMEM((_NL * _BW, DIM), jnp.float32),
            pltpu.VMEM((_NL * _BW, DIM), jnp.float32),
            pltpu.VMEM((_NL * 4 * 1024,), jnp.float32),
            pltpu.VMEM((_NL * 4 * 1024,), jnp.float32),
            pltpu.SemaphoreType.DMA,
            pltpu.SemaphoreType.DMA,
        ],
        compiler_params=pltpu.CompilerParams(use_tc_tiling_on_sc=False,
                                             needs_layout_passes=False),
    )(post_t, resp_t, wiki_t, table)


def kernel(post, resp, wiki, table):
    outs = _embed3(jnp.transpose(post), jnp.transpose(resp),
                   jnp.transpose(wiki), table)
    return tuple(
        o.reshape(L, 4, _NW, 8, _BW).transpose(2, 4, 0, 1, 3)
        .reshape(B, L, DIM) for o in outs)


# speedup vs baseline: 1.9242x; 1.4392x over previous
import functools

import jax
import jax.numpy as jnp
from jax import lax
from jax.experimental import pallas as pl
from jax.experimental.pallas import tpu as pltpu
from jax.experimental.pallas import tpu_sc as plsc

VOCAB = 1000000
DIM = 32
B = 4096
L = 50

_info = plsc.get_sparse_core_info()
_NC = _info.num_cores
_NS = _info.num_subcores
_NW = _NC * _NS
_BW = B // _NW
_NL = 5
_NJ = L // _NL


def _transpose_chunk(rows2, tile2, lane16):
    def body(g, carry):
        for u in range(4):
            r = g * 4 + u
            li = r >> 7
            bl = r & 127
            row0 = lane16 + li * 32
            col = jnp.full((16,), bl, jnp.int32)
            v0 = rows2[r, pl.ds(0, 16)]
            v1 = rows2[r, pl.ds(16, 16)]
            plsc.store_scatter(tile2, [row0, col], v0)
            plsc.store_scatter(tile2, [row0 + 16, col], v1)
        return carry
    lax.fori_loop(0, (_NL * _BW) // 4, body, 0)


def _gather_kernel(post_i, resp_i, wiki_i, table, post_o, resp_o, wiki_o,
                   idx_v, rows_a, rows_b, tile_a, tile_b, sem_g, sem_w):
    rows_v = (rows_a, rows_b)
    tile_v = (tile_a, tile_b)
    wid = lax.axis_index("s") * _NC + lax.axis_index("c")
    b0 = wid * _BW
    lane16 = lax.iota(jnp.int32, 16)
    streams = ((post_i, post_o), (resp_i, resp_o), (wiki_i, wiki_o))
    jobs = [(s, l0) for s in range(3) for l0 in range(0, L, _NL)]
    n = len(jobs)

    def load_idx(j, b):
        s, l0 = jobs[j]
        pltpu.sync_copy(streams[s][0].at[pl.ds(l0, _NL), pl.ds(b0, _BW)],
                        idx_v.at[b])

    def fire_gathers(j, b):
        return [pltpu.async_copy(table.at[idx_v.at[b, li]],
                                 rows_v[b].at[pl.ds(li * _BW, _BW)], sem_g)
                for li in range(_NL)]

    def fire_writebacks(j, b):
        s, l0 = jobs[j]
        out = streams[s][1]
        tile = tile_v[b]

        def body(t, carry):
            pltpu.async_copy(tile.at[pl.ds(t * 8, 8), pl.ds(0, _BW)],
                             out.at[l0 + (t >> 2), t & 3, wid], sem_w)
            return carry
        lax.fori_loop(0, _NL * 4, body, 0)

    def drain_writebacks(k):
        def body(t, carry):
            pltpu.make_async_copy(
                tile_a.at[pl.ds(0, 8), pl.ds(0, _BW)],
                post_o.at[0, 0, wid], sem_w).wait()
            return carry
        lax.fori_loop(0, k, body, 0)

    gat = [None, None]
    for j in range(n):
        b = j & 1
        if j >= 2:
            drain_writebacks(_NL * 4)
        load_idx(j, b)
        gat[b] = fire_gathers(j, b)
        if j > 0:
            pb = 1 - b
            for cp in gat[pb]:
                cp.wait()
            _transpose_chunk(rows_v[pb], tile_v[pb], lane16)
            fire_writebacks(j - 1, pb)
    lb = (n - 1) & 1
    for cp in gat[lb]:
        cp.wait()
    _transpose_chunk(rows_v[lb], tile_v[lb], lane16)
    fire_writebacks(n - 1, lb)
    drain_writebacks(2 * _NL * 4)


@jax.jit
def _embed3(post_t, resp_t, wiki_t, table):
    mesh = plsc.VectorSubcoreMesh(core_axis_name="c", subcore_axis_name="s")
    out = jax.ShapeDtypeStruct((L, 4, _NW, 8, _BW), jnp.float32)
    return pl.kernel(
        _gather_kernel,
        mesh=mesh,
        out_type=(out, out, out),
        scratch_types=[
            pltpu.VMEM((2, _NL, _BW), jnp.int32),
            pltpu.VMEM((_NL * _BW, DIM), jnp.float32),
            pltpu.VMEM((_NL * _BW, DIM), jnp.float32),
            pltpu.VMEM((_NL * 4 * 8, 129), jnp.float32),
            pltpu.VMEM((_NL * 4 * 8, 129), jnp.float32),
            pltpu.SemaphoreType.DMA,
            pltpu.SemaphoreType.DMA,
        ],
        compiler_params=pltpu.CompilerParams(use_tc_tiling_on_sc=False,
                                             needs_layout_passes=False),
    )(post_t, resp_t, wiki_t, table)


def kernel(post, resp, wiki, table):
    outs = _embed3(jnp.transpose(post), jnp.transpose(resp),
                   jnp.transpose(wiki), table)
    return tuple(
        o.transpose(2, 4, 0, 1, 3).reshape(B, L, DIM) for o in outs)


# speedup vs baseline: 1.9851x; 1.0317x over previous
import functools

import jax
import jax.numpy as jnp
from jax import lax
from jax.experimental import pallas as pl
from jax.experimental.pallas import tpu as pltpu
from jax.experimental.pallas import tpu_sc as plsc

VOCAB = 1000000
DIM = 32
B = 4096
L = 50

_info = plsc.get_sparse_core_info()
_NC = _info.num_cores
_NS = _info.num_subcores
_NW = _NC * _NS
_BW = B // _NW
_NL = 5
_NJ = L // _NL


def _transpose_chunk(rows2, tile2, lane16):
    def body(g, carry):
        for u in range(4):
            r = g * 4 + u
            li = r >> 7
            bl = r & 127
            row0 = lane16 + li * 32
            col = jnp.full((16,), bl, jnp.int32)
            v0 = rows2[r, pl.ds(0, 16)]
            v1 = rows2[r, pl.ds(16, 16)]
            plsc.store_scatter(tile2, [row0, col], v0)
            plsc.store_scatter(tile2, [row0 + 16, col], v1)
        return carry
    lax.fori_loop(0, (_NL * _BW) // 4, body, 0)


def _gather_kernel(post_i, resp_i, wiki_i, table, post_o, resp_o, wiki_o,
                   idx_v, rows_a, rows_b, tile_a, tile_b, sem_g, sem_w,
                   sem_i):
    rows_v = (rows_a, rows_b)
    tile_v = (tile_a, tile_b)
    wid = lax.axis_index("s") * _NC + lax.axis_index("c")
    b0 = wid * _BW
    lane16 = lax.iota(jnp.int32, 16)
    streams = ((post_i, post_o), (resp_i, resp_o), (wiki_i, wiki_o))
    jobs = [(s, l0) for s in range(3) for l0 in range(0, L, _NL)]
    n = len(jobs)

    def start_idx(j, b):
        s, l0 = jobs[j]
        return pltpu.async_copy(
            streams[s][0].at[pl.ds(l0, _NL), pl.ds(b0, _BW)],
            idx_v.at[b], sem_i)

    def fire_gathers(j, b):
        return [pltpu.async_copy(table.at[idx_v.at[b, li]],
                                 rows_v[b].at[pl.ds(li * _BW, _BW)], sem_g)
                for li in range(_NL)]

    def fire_writebacks(j, b):
        s, l0 = jobs[j]
        out = streams[s][1]
        tile = tile_v[b]

        def body(t, carry):
            pltpu.async_copy(tile.at[pl.ds(t * 8, 8), pl.ds(0, _BW)],
                             out.at[l0 + (t >> 2), t & 3, wid], sem_w)
            return carry
        lax.fori_loop(0, _NL * 4, body, 0)

    def drain_writebacks(k):
        def body(t, carry):
            pltpu.make_async_copy(
                tile_a.at[pl.ds(0, 8), pl.ds(0, _BW)],
                post_o.at[0, 0, wid], sem_w).wait()
            return carry
        lax.fori_loop(0, k, body, 0)

    gat = [None, None]
    idxc = [None, None]
    idxc[0] = start_idx(0, 0)
    idxc[0].wait()
    gat[0] = fire_gathers(0, 0)
    idxc[1] = start_idx(1, 1)
    for j in range(1, n):
        b = j & 1
        pb = 1 - b
        if j >= 2:
            drain_writebacks(_NL * 4)
        for cp in gat[pb]:
            cp.wait()
        idxc[b].wait()
        gat[b] = fire_gathers(j, b)
        if j + 1 < n:
            idxc[pb] = start_idx(j + 1, pb)
        _transpose_chunk(rows_v[pb], tile_v[pb], lane16)
        fire_writebacks(j - 1, pb)
    lb = (n - 1) & 1
    for cp in gat[lb]:
        cp.wait()
    _transpose_chunk(rows_v[lb], tile_v[lb], lane16)
    fire_writebacks(n - 1, lb)
    drain_writebacks(2 * _NL * 4)


@jax.jit
def _embed3(post_t, resp_t, wiki_t, table):
    mesh = plsc.VectorSubcoreMesh(core_axis_name="c", subcore_axis_name="s")
    out = jax.ShapeDtypeStruct((L, 4, _NW, 8, _BW), jnp.float32)
    return pl.kernel(
        _gather_kernel,
        mesh=mesh,
        out_type=(out, out, out),
        scratch_types=[
            pltpu.VMEM((2, _NL, _BW), jnp.int32),
            pltpu.VMEM((_NL * _BW, DIM), jnp.float32),
            pltpu.VMEM((_NL * _BW, DIM), jnp.float32),
            pltpu.VMEM((_NL * 4 * 8, 129), jnp.float32),
            pltpu.VMEM((_NL * 4 * 8, 129), jnp.float32),
            pltpu.SemaphoreType.DMA,
            pltpu.SemaphoreType.DMA,
            pltpu.SemaphoreType.DMA,
        ],
        compiler_params=pltpu.CompilerParams(use_tc_tiling_on_sc=False,
                                             needs_layout_passes=False),
    )(post_t, resp_t, wiki_t, table)


def kernel(post, resp, wiki, table):
    outs = _embed3(jnp.transpose(post), jnp.transpose(resp),
                   jnp.transpose(wiki), table)
    return tuple(
        o.transpose(2, 4, 0, 1, 3).reshape(B, L, DIM) for o in outs)
